# baseline (speedup 1.0000x reference)
import functools

import jax
import jax.numpy as jnp
from jax import lax
from jax.experimental import pallas as pl
from jax.experimental.pallas import tpu as pltpu
from jax.experimental.pallas import tpu_sc as plsc


def kernel(syndrome, table_keys, table_vals, table_occ, hash_coeffs):
    del table_occ
    B, D = syndrome.shape
    M = table_keys.shape[0]
    info = plsc.get_sparse_core_info()
    NC, NS, L = info.num_cores, info.num_subcores, info.num_lanes
    NW = NC * NS
    b_per_w = B // NW
    q_per_c = B // NC
    f_per_w = D // NS
    TRB = B // 128
    HB = M // 128

    def as_x(a, n):
        return a.reshape(n // 128, 128, D // 8, 8).transpose(2, 0, 3, 1)

    @functools.partial(
        pl.kernel,
        mesh=plsc.VectorSubcoreMesh(core_axis_name="c", subcore_axis_name="s"),
        out_type=jax.ShapeDtypeStruct((D // 8, TRB, 8, 128), jnp.float32),
        compiler_params=pltpu.CompilerParams(
            needs_layout_passes=False,
            use_tc_tiling_on_sc=False,
            skip_device_barrier=True,
        ),
        scratch_types=[
            pltpu.VMEM((8, 8, 128), jnp.float32),
            pltpu.VMEM((D,), jnp.int32),
            pltpu.VMEM((b_per_w,), jnp.int32),
            pltpu.VMEM((q_per_c,), jnp.int32),
            pltpu.VMEM((q_per_c,), jnp.int32),
            pltpu.VMEM((q_per_c,), jnp.int32),
            pltpu.VMEM((128, 128), jnp.float32),
            pltpu.VMEM((128, 128), jnp.float32),
            pltpu.VMEM((4, 16, 128), jnp.float32),
            pltpu.VMEM((q_per_c,), jnp.int32),
            pltpu.VMEM((16, q_per_c), jnp.int32),
            pltpu.VMEM((q_per_c,), jnp.float32),
            pltpu.VMEM((16, 128), jnp.float32),
            pltpu.VMEM_SHARED((q_per_c,), jnp.int32),
            pltpu.VMEM_SHARED((16, q_per_c), jnp.int32),
            pltpu.SemaphoreType.DMA,
            pltpu.SemaphoreType.DMA,
            pltpu.SemaphoreType.DMA,
            pltpu.SemaphoreType.DMA,
        ],
    )
    def sc_kernel(syn_hbm, keys_hbm, vals_hbm, coef_hbm, out_hbm,
                  syn_v, coef_v, h_v, hall_v, hb_v, hi_v, tab_a, tab_b,
                  synd_v, mm_v, mmall_v, hit_v, od_v, h_sh, mm_sh,
                  sem_a, sem_b, sem_s, sem_d):
        c = lax.axis_index("c")
        s = lax.axis_index("s")
        tr = c * NS + s
        tabs = (tab_a, tab_b)
        sems = (sem_a, sem_b)
        feats = [s * f_per_w + j for j in range(f_per_w)]

        cp_syn = pltpu.async_copy(syn_hbm.at[:, tr], syn_v, sem_s)
        cps_sd = [
            pltpu.async_copy(
                syn_hbm.at[d // 8, pl.ds(c * NS, NS), d % 8],
                synd_v.at[j], sem_d)
            for j, d in enumerate(feats)
        ]
        cp_k = [
            pltpu.async_copy(keys_hbm.at[d // 8, :, d % 8], tabs[j % 2],
                             sems[j % 2])
            for j, d in enumerate(feats[:2])
        ]
        pltpu.sync_copy(coef_hbm, coef_v)
        cp_syn.wait()

        def hash_body(g, carry):
            acc = jnp.zeros((L,), jnp.int32)
            for d in range(D):
                sd = syn_v[d // 8, d % 8, pl.ds(g * L, L)].astype(jnp.int32)
                cf = coef_v[pl.ds((d // L) * L, L)][d % L]
                acc = acc + sd * cf
            if M & (M - 1) == 0:
                h = lax.bitwise_and(acc, M - 1)
            else:
                h = lax.rem(acc, M)
            h_v[pl.ds(g * L, L)] = h
            return carry

        lax.fori_loop(0, b_per_w // L, hash_body, 0)
        pltpu.sync_copy(h_v, h_sh.at[pl.ds(s * b_per_w, b_per_w)])
        plsc.subcore_barrier()
        pltpu.sync_copy(h_sh, hall_v)

        def split_body(q, carry):
            sl = pl.ds(q * L, L)
            h16 = hall_v[sl]
            hb_v[sl] = lax.shift_right_logical(h16, 7)
            hi_v[sl] = lax.bitwise_and(h16, 127)
            return carry

        lax.fori_loop(0, q_per_c // L, split_body, 0)

        for cp in cps_sd:
            cp.wait()

        def feat_compare(j, first):
            def cmp_body(qb, carry):
                for g in range(8):
                    sl = pl.ds(qb * 128 + g * L, L)
                    kd = plsc.load_gather(tabs[j % 2], [hb_v[sl], hi_v[sl]])
                    sd = synd_v[j, qb, pl.ds(g * L, L)]
                    ne = (kd != sd).astype(jnp.int32)
                    if first:
                        mm_v[sl] = ne
                    else:
                        mm_v[sl] = mm_v[sl] + ne
                return carry

            lax.fori_loop(0, q_per_c // 128, cmp_body, 0)

        for j in range(f_per_w):
            cp_k[j].wait()
            feat_compare(j, j == 0)
            if j + 2 < f_per_w:
                cp_k.append(pltpu.async_copy(
                    keys_hbm.at[feats[j + 2] // 8, :, feats[j + 2] % 8],
                    tabs[j % 2], sems[j % 2]))
        cp_v = [
            pltpu.async_copy(vals_hbm.at[d // 8, :, d % 8], tabs[j % 2],
                             sems[j % 2])
            for j, d in enumerate(feats[:2])
        ]

        pltpu.sync_copy(mm_v, mm_sh.at[s])
        plsc.subcore_barrier()
        pltpu.sync_copy(mm_sh, mmall_v)

        def hit_body(q, carry):
            sl = pl.ds(q * L, L)
            t = mmall_v[0, sl]
            for w in range(1, NS):
                t = t + mmall_v[w, sl]
            hit_v[sl] = jnp.where(t == 0, jnp.full((L,), 1.0, jnp.float32),
                                  jnp.zeros((L,), jnp.float32))
            return carry

        lax.fori_loop(0, q_per_c // L, hit_body, 0)

        def feat_output(j, d):
            def out_body(qb, carry):
                for g in range(8):
                    sl = pl.ds(qb * 128 + g * L, L)
                    vd = plsc.load_gather(tabs[j % 2], [hb_v[sl], hi_v[sl]])
                    od_v[qb, pl.ds(g * L, L)] = vd * hit_v[sl]
                return carry

            lax.fori_loop(0, q_per_c // 128, out_body, 0)
            pltpu.sync_copy(od_v, out_hbm.at[d // 8, pl.ds(c * NS, NS), d % 8])

        for j in range(f_per_w):
            cp_v[j].wait()
            feat_output(j, feats[j])
            if j + 2 < f_per_w:
                cp_v.append(pltpu.async_copy(
                    vals_hbm.at[feats[j + 2] // 8, :, feats[j + 2] % 8],
                    tabs[j % 2], sems[j % 2]))

    out4 = sc_kernel(as_x(syndrome, B), as_x(table_keys, M),
                     as_x(table_vals, M), hash_coeffs)
    return out4.transpose(1, 3, 0, 2).reshape(B, D)


# speedup vs baseline: 1.0470x; 1.0470x over previous
import functools

import jax
import jax.numpy as jnp
from jax import lax
from jax.experimental import pallas as pl
from jax.experimental.pallas import tpu as pltpu
from jax.experimental.pallas import tpu_sc as plsc


def kernel(syndrome, table_keys, table_vals, table_occ, hash_coeffs):
    del table_occ
    B, D = syndrome.shape
    M = table_keys.shape[0]
    info = plsc.get_sparse_core_info()
    NC, NS, L = info.num_cores, info.num_subcores, info.num_lanes
    NW = NC * NS
    b_per_w = B // NW
    q_per_c = B // NC
    f_per_w = D // NS
    TRB = B // 128
    HB = M // 128

    def as_x(a, n):
        return a.reshape(n // 128, 128, D // 8, 8).transpose(2, 0, 3, 1)

    @functools.partial(
        pl.kernel,
        mesh=plsc.VectorSubcoreMesh(core_axis_name="c", subcore_axis_name="s"),
        out_type=jax.ShapeDtypeStruct((D // 8, TRB, 8, 128), jnp.float32),
        compiler_params=pltpu.CompilerParams(
            needs_layout_passes=False,
            use_tc_tiling_on_sc=False,
            skip_device_barrier=True,
        ),
        scratch_types=[
            pltpu.VMEM((8, 8, 128), jnp.float32),
            pltpu.VMEM((D,), jnp.int32),
            pltpu.VMEM((b_per_w,), jnp.int32),
            pltpu.VMEM((q_per_c,), jnp.int32),
            pltpu.VMEM((128, 128), jnp.float32),
            pltpu.VMEM((128, 128), jnp.float32),
            pltpu.VMEM((4, 16, 128), jnp.float32),
            pltpu.VMEM((q_per_c,), jnp.int32),
            pltpu.VMEM((16, q_per_c), jnp.int32),
            pltpu.VMEM((q_per_c,), jnp.float32),
            pltpu.VMEM((16, 128), jnp.float32),
            pltpu.VMEM_SHARED((q_per_c,), jnp.int32),
            pltpu.VMEM_SHARED((16, q_per_c), jnp.int32),
            pltpu.SemaphoreType.DMA,
            pltpu.SemaphoreType.DMA,
            pltpu.SemaphoreType.DMA,
            pltpu.SemaphoreType.DMA,
        ],
    )
    def sc_kernel(syn_hbm, keys_hbm, vals_hbm, coef_hbm, out_hbm,
                  syn_v, coef_v, h_v, hall_v, tab_a, tab_b,
                  synd_v, mm_v, mmall_v, hit_v, od_v, h_sh, mm_sh,
                  sem_a, sem_b, sem_s, sem_d):
        c = lax.axis_index("c")
        s = lax.axis_index("s")
        tr = c * NS + s
        tabs = (tab_a, tab_b)
        sems = (sem_a, sem_b)
        feats = [s * f_per_w + j for j in range(f_per_w)]

        cp_syn = pltpu.async_copy(syn_hbm.at[:, tr], syn_v, sem_s)
        cps_sd = [
            pltpu.async_copy(
                syn_hbm.at[d // 8, pl.ds(c * NS, NS), d % 8],
                synd_v.at[j], sem_d)
            for j, d in enumerate(feats)
        ]
        cp_k = [
            pltpu.async_copy(keys_hbm.at[d // 8, :, d % 8], tabs[j % 2],
                             sems[j % 2])
            for j, d in enumerate(feats[:2])
        ]
        pltpu.sync_copy(coef_hbm, coef_v)
        cp_syn.wait()

        def hash_body(g, carry):
            acc = jnp.zeros((L,), jnp.int32)
            for d in range(D):
                sd = syn_v[d // 8, d % 8, pl.ds(g * L, L)].astype(jnp.int32)
                cf = coef_v[pl.ds((d // L) * L, L)][d % L]
                acc = acc + sd * cf
            if M & (M - 1) == 0:
                h = lax.bitwise_and(acc, M - 1)
            else:
                h = lax.rem(acc, M)
            h_v[pl.ds(g * L, L)] = h
            return carry

        lax.fori_loop(0, b_per_w // L, hash_body, 0)
        pltpu.sync_copy(h_v, h_sh.at[pl.ds(s * b_per_w, b_per_w)])
        plsc.subcore_barrier()
        pltpu.sync_copy(h_sh, hall_v)

        for cp in cps_sd:
            cp.wait()

        def feat_compare(j, first):
            def cmp_body(qb, carry):
                for g in range(8):
                    sl = pl.ds(qb * 128 + g * L, L)
                    h16 = hall_v[sl]
                    kd = plsc.load_gather(
                        tabs[j % 2], [lax.shift_right_logical(h16, 7),
                                      lax.bitwise_and(h16, 127)])
                    sd = synd_v[j, qb, pl.ds(g * L, L)]
                    ne = (kd != sd).astype(jnp.int32)
                    if first:
                        mm_v[sl] = ne
                    else:
                        mm_v[sl] = mm_v[sl] + ne
                return carry

            lax.fori_loop(0, q_per_c // 128, cmp_body, 0)

        for j in range(f_per_w):
            cp_k[j].wait()
            feat_compare(j, j == 0)
            if j + 2 < f_per_w:
                cp_k.append(pltpu.async_copy(
                    keys_hbm.at[feats[j + 2] // 8, :, feats[j + 2] % 8],
                    tabs[j % 2], sems[j % 2]))
        cp_v = [
            pltpu.async_copy(vals_hbm.at[d // 8, :, d % 8], tabs[j % 2],
                             sems[j % 2])
            for j, d in enumerate(feats[:2])
        ]

        pltpu.sync_copy(mm_v, mm_sh.at[s])
        plsc.subcore_barrier()
        pltpu.sync_copy(mm_sh, mmall_v)

        def hit_body(q, carry):
            sl = pl.ds(q * L, L)
            t = mmall_v[0, sl]
            for w in range(1, NS):
                t = t + mmall_v[w, sl]
            hit_v[sl] = jnp.where(t == 0, jnp.full((L,), 1.0, jnp.float32),
                                  jnp.zeros((L,), jnp.float32))
            return carry

        lax.fori_loop(0, q_per_c // L, hit_body, 0)

        def feat_output(j, d):
            def out_body(qb, carry):
                for g in range(8):
                    sl = pl.ds(qb * 128 + g * L, L)
                    h16 = hall_v[sl]
                    vd = plsc.load_gather(
                        tabs[j % 2], [lax.shift_right_logical(h16, 7),
                                      lax.bitwise_and(h16, 127)])
                    od_v[qb, pl.ds(g * L, L)] = vd * hit_v[sl]
                return carry

            lax.fori_loop(0, q_per_c // 128, out_body, 0)
            pltpu.sync_copy(od_v, out_hbm.at[d // 8, pl.ds(c * NS, NS), d % 8])

        for j in range(f_per_w):
            cp_v[j].wait()
            feat_output(j, feats[j])
            if j + 2 < f_per_w:
                cp_v.append(pltpu.async_copy(
                    vals_hbm.at[feats[j + 2] // 8, :, feats[j + 2] % 8],
                    tabs[j % 2], sems[j % 2]))

    out4 = sc_kernel(as_x(syndrome, B), as_x(table_keys, M),
                     as_x(table_vals, M), hash_coeffs)
    return out4.transpose(1, 3, 0, 2).reshape(B, D)
